# baseline (speedup 1.0000x reference)
import functools

import jax
import jax.numpy as jnp
from jax import lax
from jax.experimental import pallas as pl
from jax.experimental.pallas import tpu as pltpu
from jax.experimental.pallas import tpu_sc as plsc

NUM_GRAPHS = 64

_NC = 2
_NS = 16
_NW = _NC * _NS


_CHUNK = 104
_NBUF = 2
_NPAD = 8


def _pad_edges(src, dst, n_nodes):
    e = src.shape[0]
    epw = e // _NW
    grp = _CHUNK * _NBUF
    epw_pad = -(-epw // grp) * grp
    pad = epw_pad - epw
    src2 = src.reshape(_NW, epw)
    dst2 = dst.reshape(_NW, epw)
    if pad:
        src2 = jnp.pad(src2, ((0, 0), (0, pad)))
        dummy = (jnp.arange(pad, dtype=jnp.int32) % _NPAD) + n_nodes
        dst2 = jnp.concatenate(
            [dst2, jnp.broadcast_to(dummy, (_NW, pad))], axis=1)
    nch = epw_pad // _CHUNK
    return src2.reshape(_NW, nch, _CHUNK), dst2.reshape(_NW, nch, _CHUNK), nch


def _make_edge_agg(n_nodes, feat, nchunks, zrows=24):
    n_acc = n_nodes + _NPAD
    rpt = (n_nodes // _NS) // 8 * 8
    rem = n_nodes - rpt * _NS
    zrem = n_acc - rpt * _NS
    assert rpt % zrows == 0 and rem % 8 == 0 and zrem <= zrows
    assert nchunks % _NBUF == 0
    ngroups = nchunks // _NBUF

    mesh = plsc.VectorSubcoreMesh(core_axis_name="c", subcore_axis_name="s")

    @functools.partial(
        pl.kernel,
        mesh=mesh,
        compiler_params=pltpu.CompilerParams(use_tc_tiling_on_sc=False),
        out_type=jax.ShapeDtypeStruct((_NC, n_nodes, feat), jnp.float32),
        scratch_types=(
            [pltpu.VMEM((nchunks, _CHUNK), jnp.int32),
             pltpu.VMEM((nchunks, _CHUNK), jnp.int32)]
            + [pltpu.VMEM((_CHUNK, feat), jnp.float32)] * _NBUF
            + [pltpu.VMEM((zrows, feat), jnp.float32),
               pltpu.VMEM_SHARED((n_acc, feat), jnp.float32)]
            + [pltpu.SemaphoreType.DMA] * (2 * _NBUF)
        ),
    )
    def edge_agg(y_hbm, src_hbm, dst_hbm, out_hbm, *scr):
        srcw, dstw = scr[0], scr[1]
        rows = scr[2:2 + _NBUF]
        zero_v, acc = scr[2 + _NBUF], scr[3 + _NBUF]
        gsem = scr[4 + _NBUF:4 + 2 * _NBUF]
        ssem = scr[4 + 2 * _NBUF:4 + 3 * _NBUF]

        cid = lax.axis_index("c")
        sid = lax.axis_index("s")
        wid = sid * _NC + cid
        is_last = sid == (_NS - 1)

        pltpu.sync_copy(src_hbm.at[wid], srcw)
        pltpu.sync_copy(dst_hbm.at[wid], dstw)

        @pl.loop(0, zrows)
        def _(i):
            @pl.loop(0, feat // 16)
            def _(j):
                zero_v[i, pl.ds(j * 16, 16)] = jnp.zeros((16,), jnp.float32)

        row0 = sid * rpt

        @pl.loop(0, rpt // zrows)
        def _(k):
            pltpu.sync_copy(zero_v, acc.at[pl.ds(row0 + k * zrows, zrows)])

        @pl.when(is_last)
        def _():
            pltpu.sync_copy(zero_v.at[pl.ds(0, zrem)],
                            acc.at[pl.ds(_NS * rpt, zrem)])

        plsc.subcore_barrier()

        def wait_on(sem, b):
            pltpu.make_async_copy(y_hbm.at[pl.ds(0, _CHUNK)], rows[b],
                                  sem).wait()

        for b in range(_NBUF):
            pltpu.async_copy(y_hbm.at[srcw.at[b]], rows[b], gsem[b])

        @pl.loop(0, ngroups - 1)
        def _(g):
            i0 = g * _NBUF
            for b in range(_NBUF):
                wait_on(gsem[b], b)
                pltpu.async_copy(rows[b], acc.at[dstw.at[i0 + b]],
                                 ssem[b], add=True)
            for b in range(_NBUF):
                wait_on(ssem[b], b)
                pltpu.async_copy(y_hbm.at[srcw.at[i0 + _NBUF + b]],
                                 rows[b], gsem[b])

        i0 = nchunks - _NBUF
        for b in range(_NBUF):
            wait_on(gsem[b], b)
            pltpu.async_copy(rows[b], acc.at[dstw.at[i0 + b]],
                             ssem[b], add=True)
        for b in range(_NBUF):
            wait_on(ssem[b], b)

        plsc.subcore_barrier()
        pltpu.sync_copy(acc.at[pl.ds(row0, rpt)],
                        out_hbm.at[cid, pl.ds(row0, rpt)])
        if rem:
            @pl.when(is_last)
            def _():
                pltpu.sync_copy(acc.at[pl.ds(_NS * rpt, rem)],
                                out_hbm.at[cid, pl.ds(_NS * rpt, rem)])

    return edge_agg


def _bdot(a, b):
    return jnp.dot(a.astype(jnp.bfloat16), b.astype(jnp.bfloat16),
                   preferred_element_type=jnp.float32)


def _xdot(a, b, dn=None):
    if dn is None:
        return jnp.dot(a, b, preferred_element_type=jnp.float32,
                       precision=lax.Precision.HIGHEST)
    return lax.dot_general(a, b, dimension_numbers=dn,
                           preferred_element_type=jnp.float32,
                           precision=lax.Precision.HIGHEST)


def _layer_body(h_ref, agg_ref, wa_ref, ba_ref, wb_ref, bb_ref, out_ref):
    z = h_ref[...] + agg_ref[0] + agg_ref[1]
    t = jnp.maximum(_bdot(z, wa_ref[...]) + ba_ref[...], 0.0)
    out_ref[...] = jnp.maximum(_bdot(t, wb_ref[...]) + bb_ref[...], 0.0)


def _tc_layer(h, aggp, wa, ba, wb, bb):
    n = h.shape[0]
    return pl.pallas_call(
        _layer_body,
        out_shape=jax.ShapeDtypeStruct((n, wb.shape[1]), jnp.float32),
    )(h, aggp, wa, ba.reshape(1, -1), wb, bb.reshape(1, -1))


def _final_body(h1_ref, agg_ref, w3_ref, b3_ref, w4_ref, b4_ref, wg_ref,
                bg_ref, batch_ref, wc1_ref, bc1_ref, wc2_ref, bc2_ref,
                wr1_ref, br1_ref, wr2_ref, br2_ref, cls_ref, reg_ref):
    n = h1_ref.shape[0]
    z = h1_ref[...] + agg_ref[0] + agg_ref[1]
    t = jnp.maximum(_bdot(z, w3_ref[...]) + b3_ref[...], 0.0)
    h2 = jnp.maximum(_bdot(t, w4_ref[...]) + b4_ref[...], 0.0)
    gate = _bdot(h2, wg_ref[...]) + bg_ref[...]

    batch = batch_ref[...]
    gids = lax.broadcasted_iota(jnp.int32, (n, NUM_GRAPHS), 1)
    mask = (batch == gids)
    maskf = mask.astype(jnp.float32)

    neg = jnp.float32(-1e30)
    gm = jnp.where(mask, gate, neg)
    m = jnp.max(gm, axis=0, keepdims=True)
    mb = jnp.sum(jnp.where(mask, m, 0.0), axis=1, keepdims=True)
    e = jnp.exp(gate - mb)
    dn = (((0,), (0,)), ((), ()))
    denom = _xdot(maskf, e, dn)
    denb = _xdot(maskf, denom)
    w = e / denb
    g = _xdot(maskf, w * h2, dn)

    c1 = jnp.maximum(_bdot(g, wc1_ref[...]) + bc1_ref[...], 0.0)
    cls_ref[...] = _bdot(c1, wc2_ref[...]) + bc2_ref[...]
    r1 = jnp.maximum(_bdot(g, wr1_ref[...]) + br1_ref[...], 0.0)
    reg_ref[...] = _bdot(r1, wr2_ref[...]) + br2_ref[...]


def _tc_final(h1, aggp, w3, b3, w4, b4, wg, bg, batch, wc1, bc1, wc2, bc2,
              wr1, br1, wr2, br2):
    c = wc2.shape[1]
    return pl.pallas_call(
        _final_body,
        out_shape=[jax.ShapeDtypeStruct((NUM_GRAPHS, c), jnp.float32),
                   jax.ShapeDtypeStruct((NUM_GRAPHS, 1), jnp.float32)],
    )(h1, aggp, w3, b3.reshape(1, -1), w4, b4.reshape(1, -1), wg,
      bg.reshape(1, -1), batch.reshape(-1, 1), wc1, bc1.reshape(1, -1),
      wc2, bc2.reshape(1, -1), wr1, br1.reshape(1, -1), wr2,
      br2.reshape(1, -1))


def kernel(x, edge_index, batch, W1, b1, W2, b2, W3, b3, W4, b4, Wg, bg,
           Wc1, bc1, Wc2, bc2, Wr1, br1, Wr2, br2):
    n, d = x.shape
    h = W1.shape[1]
    src = edge_index[0]
    dst = edge_index[1]
    src3, dst3, nch = _pad_edges(src, dst, n)

    agg1 = _make_edge_agg(n, d, nch)(x, src3, dst3)
    h1 = _tc_layer(x, agg1, W1, b1, W2, b2)
    agg2 = _make_edge_agg(n, h, nch)(h1, src3, dst3)
    cls, reg = _tc_final(h1, agg2, W3, b3, W4, b4, Wg, bg, batch,
                         Wc1, bc1, Wc2, bc2, Wr1, br1, Wr2, br2)
    return (cls, reg)


# speedup vs baseline: 1.4055x; 1.4055x over previous
import functools

import jax
import jax.numpy as jnp
from jax import lax
from jax.experimental import pallas as pl
from jax.experimental.pallas import tpu as pltpu
from jax.experimental.pallas import tpu_sc as plsc

NUM_GRAPHS = 64

_NC = 2
_NS = 16
_NW = _NC * _NS


_CHUNK = 80
_NBUF = 2
_NPAD = 8


def _pad_edges(src, dst, n_nodes):
    e = src.shape[0]
    epw = e // _NW
    grp = _CHUNK * _NBUF
    epw_pad = -(-epw // grp) * grp
    pad = epw_pad - epw
    src2 = src.reshape(_NW, epw)
    dst2 = dst.reshape(_NW, epw)
    if pad:
        src2 = jnp.pad(src2, ((0, 0), (0, pad)))
        dummy = (jnp.arange(pad, dtype=jnp.int32) % _NPAD) + n_nodes
        dst2 = jnp.concatenate(
            [dst2, jnp.broadcast_to(dummy, (_NW, pad))], axis=1)
    nch = epw_pad // _CHUNK
    return src2.reshape(_NW, nch, _CHUNK), dst2.reshape(_NW, nch, _CHUNK), nch


def _make_edge_agg(n_nodes, feat, nchunks, zrows=24):
    n_acc = n_nodes + _NPAD
    rpt = (n_nodes // _NS) // 8 * 8
    rem = n_nodes - rpt * _NS
    zrem = n_acc - rpt * _NS
    assert rpt % zrows == 0 and rem % 8 == 0 and zrem <= zrows
    assert nchunks % _NBUF == 0
    ngroups = nchunks // _NBUF

    mesh = plsc.VectorSubcoreMesh(core_axis_name="c", subcore_axis_name="s")

    @functools.partial(
        pl.kernel,
        mesh=mesh,
        compiler_params=pltpu.CompilerParams(use_tc_tiling_on_sc=False),
        out_type=jax.ShapeDtypeStruct((_NC, n_nodes, feat), jnp.float32),
        scratch_types=(
            [pltpu.VMEM((nchunks, _CHUNK), jnp.int32),
             pltpu.VMEM((nchunks, _CHUNK), jnp.int32)]
            + [pltpu.VMEM((_CHUNK, feat), jnp.float32)] * _NBUF
            + [pltpu.VMEM((zrows, feat), jnp.float32),
               pltpu.VMEM_SHARED((n_acc, feat), jnp.float32)]
            + [pltpu.SemaphoreType.DMA] * (2 * _NBUF)
        ),
    )
    def edge_agg(y_hbm, src_hbm, dst_hbm, out_hbm, *scr):
        srcw, dstw = scr[0], scr[1]
        rows = scr[2:2 + _NBUF]
        zero_v, acc = scr[2 + _NBUF], scr[3 + _NBUF]
        gsem = scr[4 + _NBUF:4 + 2 * _NBUF]
        ssem = scr[4 + 2 * _NBUF:4 + 3 * _NBUF]

        cid = lax.axis_index("c")
        sid = lax.axis_index("s")
        wid = sid * _NC + cid
        is_last = sid == (_NS - 1)

        pltpu.sync_copy(src_hbm.at[wid], srcw)
        pltpu.sync_copy(dst_hbm.at[wid], dstw)

        @pl.loop(0, zrows)
        def _(i):
            @pl.loop(0, feat // 16)
            def _(j):
                zero_v[i, pl.ds(j * 16, 16)] = jnp.zeros((16,), jnp.float32)

        row0 = sid * rpt

        @pl.loop(0, rpt // zrows)
        def _(k):
            pltpu.sync_copy(zero_v, acc.at[pl.ds(row0 + k * zrows, zrows)])

        @pl.when(is_last)
        def _():
            pltpu.sync_copy(zero_v.at[pl.ds(0, zrem)],
                            acc.at[pl.ds(_NS * rpt, zrem)])

        plsc.subcore_barrier()

        def wait_on(sem, b):
            pltpu.make_async_copy(y_hbm.at[pl.ds(0, _CHUNK)], rows[b],
                                  sem).wait()

        for b in range(_NBUF):
            pltpu.async_copy(y_hbm.at[srcw.at[b]], rows[b], gsem[b])

        @pl.loop(0, ngroups - 1)
        def _(g):
            i0 = g * _NBUF
            for b in range(_NBUF):
                wait_on(gsem[b], b)
                pltpu.async_copy(rows[b], acc.at[dstw.at[i0 + b]],
                                 ssem[b], add=True)
            for b in range(_NBUF):
                wait_on(ssem[b], b)
                pltpu.async_copy(y_hbm.at[srcw.at[i0 + _NBUF + b]],
                                 rows[b], gsem[b])

        i0 = nchunks - _NBUF
        for b in range(_NBUF):
            wait_on(gsem[b], b)
            pltpu.async_copy(rows[b], acc.at[dstw.at[i0 + b]],
                             ssem[b], add=True)
        for b in range(_NBUF):
            wait_on(ssem[b], b)

        plsc.subcore_barrier()
        pltpu.sync_copy(acc.at[pl.ds(row0, rpt)],
                        out_hbm.at[cid, pl.ds(row0, rpt)])
        if rem:
            @pl.when(is_last)
            def _():
                pltpu.sync_copy(acc.at[pl.ds(_NS * rpt, rem)],
                                out_hbm.at[cid, pl.ds(_NS * rpt, rem)])

    return edge_agg


def _bdot(a, b):
    return jnp.dot(a.astype(jnp.bfloat16), b.astype(jnp.bfloat16),
                   preferred_element_type=jnp.float32)


def _xdot(a, b, dn=None):
    if dn is None:
        return jnp.dot(a, b, preferred_element_type=jnp.float32,
                       precision=lax.Precision.HIGHEST)
    return lax.dot_general(a, b, dimension_numbers=dn,
                           preferred_element_type=jnp.float32,
                           precision=lax.Precision.HIGHEST)


def _layer_body(h_ref, agg_ref, wa_ref, ba_ref, wb_ref, bb_ref, out_ref):
    z = h_ref[...] + agg_ref[0] + agg_ref[1]
    t = jnp.maximum(_bdot(z, wa_ref[...]) + ba_ref[...], 0.0)
    out_ref[...] = jnp.maximum(_bdot(t, wb_ref[...]) + bb_ref[...], 0.0)


def _tc_layer(h, aggp, wa, ba, wb, bb):
    n = h.shape[0]
    return pl.pallas_call(
        _layer_body,
        out_shape=jax.ShapeDtypeStruct((n, wb.shape[1]), jnp.float32),
    )(h, aggp, wa, ba.reshape(1, -1), wb, bb.reshape(1, -1))


def _final_body(h1_ref, agg_ref, w3_ref, b3_ref, w4_ref, b4_ref, wg_ref,
                bg_ref, batch_ref, wc1_ref, bc1_ref, wc2_ref, bc2_ref,
                wr1_ref, br1_ref, wr2_ref, br2_ref, cls_ref, reg_ref):
    n = h1_ref.shape[0]
    z = h1_ref[...] + agg_ref[0] + agg_ref[1]
    t = jnp.maximum(_bdot(z, w3_ref[...]) + b3_ref[...], 0.0)
    h2 = jnp.maximum(_bdot(t, w4_ref[...]) + b4_ref[...], 0.0)
    gate = _bdot(h2, wg_ref[...]) + bg_ref[...]

    batch = batch_ref[...]
    gids = lax.broadcasted_iota(jnp.int32, (n, NUM_GRAPHS), 1)
    mask = (batch == gids)
    maskf = mask.astype(jnp.float32)

    neg = jnp.float32(-1e30)
    gm = jnp.where(mask, gate, neg)
    m = jnp.max(gm, axis=0, keepdims=True)
    mb = jnp.sum(jnp.where(mask, m, 0.0), axis=1, keepdims=True)
    e = jnp.exp(gate - mb)
    dn = (((0,), (0,)), ((), ()))
    denom = _xdot(maskf, e, dn)
    denb = _xdot(maskf, denom)
    w = e / denb
    g = _xdot(maskf, w * h2, dn)

    c1 = jnp.maximum(_bdot(g, wc1_ref[...]) + bc1_ref[...], 0.0)
    cls_ref[...] = _bdot(c1, wc2_ref[...]) + bc2_ref[...]
    r1 = jnp.maximum(_bdot(g, wr1_ref[...]) + br1_ref[...], 0.0)
    reg_ref[...] = _bdot(r1, wr2_ref[...]) + br2_ref[...]


def _tc_final(h1, aggp, w3, b3, w4, b4, wg, bg, batch, wc1, bc1, wc2, bc2,
              wr1, br1, wr2, br2):
    c = wc2.shape[1]
    return pl.pallas_call(
        _final_body,
        out_shape=[jax.ShapeDtypeStruct((NUM_GRAPHS, c), jnp.float32),
                   jax.ShapeDtypeStruct((NUM_GRAPHS, 1), jnp.float32)],
    )(h1, aggp, w3, b3.reshape(1, -1), w4, b4.reshape(1, -1), wg,
      bg.reshape(1, -1), batch.reshape(-1, 1), wc1, bc1.reshape(1, -1),
      wc2, bc2.reshape(1, -1), wr1, br1.reshape(1, -1), wr2,
      br2.reshape(1, -1))


def kernel(x, edge_index, batch, W1, b1, W2, b2, W3, b3, W4, b4, Wg, bg,
           Wc1, bc1, Wc2, bc2, Wr1, br1, Wr2, br2):
    n, d = x.shape
    h = W1.shape[1]
    src = edge_index[0]
    dst = edge_index[1]
    src3, dst3, nch = _pad_edges(src, dst, n)

    agg1 = _make_edge_agg(n, d, nch)(x, src3, dst3)
    h1 = _tc_layer(x, agg1, W1, b1, W2, b2)
    agg2 = _make_edge_agg(n, h, nch)(h1, src3, dst3)
    cls, reg = _tc_final(h1, agg2, W3, b3, W4, b4, Wg, bg, batch,
                         Wc1, bc1, Wc2, bc2, Wr1, br1, Wr2, br2)
    return (cls, reg)


# speedup vs baseline: 2.0019x; 1.4243x over previous
import functools

import jax
import jax.numpy as jnp
from jax import lax
from jax.experimental import pallas as pl
from jax.experimental.pallas import tpu as pltpu
from jax.experimental.pallas import tpu_sc as plsc

NUM_GRAPHS = 64

_NC = 2
_NS = 16
_NW = _NC * _NS


_CHUNK = 100
_NBUF = 2
_NPAD = 8


def _pad_edges(src, dst, n_nodes):
    e = src.shape[0]
    epw = e // _NW
    grp = _CHUNK * _NBUF
    epw_pad = -(-epw // grp) * grp
    pad = epw_pad - epw
    src2 = src.reshape(_NW, epw)
    dst2 = dst.reshape(_NW, epw)
    if pad:
        src2 = jnp.pad(src2, ((0, 0), (0, pad)))
        dummy = (jnp.arange(pad, dtype=jnp.int32) % _NPAD) + n_nodes
        dst2 = jnp.concatenate(
            [dst2, jnp.broadcast_to(dummy, (_NW, pad))], axis=1)
    nch = epw_pad // _CHUNK
    return src2.reshape(_NW, nch, _CHUNK), dst2.reshape(_NW, nch, _CHUNK), nch


def _make_edge_agg(n_nodes, feat, nchunks, zrows=24):
    n_acc = n_nodes + _NPAD
    rpt = (n_nodes // _NS) // 8 * 8
    rem = n_nodes - rpt * _NS
    zrem = n_acc - rpt * _NS
    assert rpt % zrows == 0 and rem % 8 == 0 and zrem <= zrows
    assert nchunks % _NBUF == 0
    ngroups = nchunks // _NBUF

    mesh = plsc.VectorSubcoreMesh(core_axis_name="c", subcore_axis_name="s")

    @functools.partial(
        pl.kernel,
        mesh=mesh,
        compiler_params=pltpu.CompilerParams(use_tc_tiling_on_sc=False),
        out_type=jax.ShapeDtypeStruct((_NC, n_nodes, feat), jnp.float32),
        scratch_types=(
            [pltpu.VMEM((nchunks, _CHUNK), jnp.int32),
             pltpu.VMEM((nchunks, _CHUNK), jnp.int32)]
            + [pltpu.VMEM((_CHUNK, feat), jnp.float32)] * _NBUF
            + [pltpu.VMEM((zrows, feat), jnp.float32),
               pltpu.VMEM_SHARED((n_acc, feat), jnp.float32)]
            + [pltpu.SemaphoreType.DMA] * (2 * _NBUF)
        ),
    )
    def edge_agg(y_hbm, src_hbm, dst_hbm, out_hbm, *scr):
        srcw, dstw = scr[0], scr[1]
        rows = scr[2:2 + _NBUF]
        zero_v, acc = scr[2 + _NBUF], scr[3 + _NBUF]
        gsem = scr[4 + _NBUF:4 + 2 * _NBUF]
        ssem = scr[4 + 2 * _NBUF:4 + 3 * _NBUF]

        cid = lax.axis_index("c")
        sid = lax.axis_index("s")
        wid = sid * _NC + cid
        is_last = sid == (_NS - 1)

        pltpu.sync_copy(src_hbm.at[wid], srcw)
        pltpu.sync_copy(dst_hbm.at[wid], dstw)

        @pl.loop(0, zrows)
        def _(i):
            @pl.loop(0, feat // 16)
            def _(j):
                zero_v[i, pl.ds(j * 16, 16)] = jnp.zeros((16,), jnp.float32)

        row0 = sid * rpt

        @pl.loop(0, rpt // zrows)
        def _(k):
            pltpu.sync_copy(zero_v, acc.at[pl.ds(row0 + k * zrows, zrows)])

        @pl.when(is_last)
        def _():
            pltpu.sync_copy(zero_v.at[pl.ds(0, zrem)],
                            acc.at[pl.ds(_NS * rpt, zrem)])

        plsc.subcore_barrier()

        def wait_on(sem, b):
            pltpu.make_async_copy(y_hbm.at[pl.ds(0, _CHUNK)], rows[b],
                                  sem).wait()

        for b in range(_NBUF):
            pltpu.async_copy(y_hbm.at[srcw.at[b]], rows[b], gsem[b])

        @pl.loop(0, ngroups - 1)
        def _(g):
            i0 = g * _NBUF
            for b in range(_NBUF):
                wait_on(gsem[b], b)
                pltpu.async_copy(rows[b], acc.at[dstw.at[i0 + b]],
                                 ssem[b], add=True)
            for b in range(_NBUF):
                wait_on(ssem[b], b)
                pltpu.async_copy(y_hbm.at[srcw.at[i0 + _NBUF + b]],
                                 rows[b], gsem[b])

        i0 = nchunks - _NBUF
        for b in range(_NBUF):
            wait_on(gsem[b], b)
            pltpu.async_copy(rows[b], acc.at[dstw.at[i0 + b]],
                             ssem[b], add=True)
        for b in range(_NBUF):
            wait_on(ssem[b], b)

        plsc.subcore_barrier()
        pltpu.sync_copy(acc.at[pl.ds(row0, rpt)],
                        out_hbm.at[cid, pl.ds(row0, rpt)])
        if rem:
            @pl.when(is_last)
            def _():
                pltpu.sync_copy(acc.at[pl.ds(_NS * rpt, rem)],
                                out_hbm.at[cid, pl.ds(_NS * rpt, rem)])

    return edge_agg


def _bdot(a, b):
    return jnp.dot(a.astype(jnp.bfloat16), b.astype(jnp.bfloat16),
                   preferred_element_type=jnp.float32)


def _xdot(a, b, dn=None):
    if dn is None:
        return jnp.dot(a, b, preferred_element_type=jnp.float32,
                       precision=lax.Precision.HIGHEST)
    return lax.dot_general(a, b, dimension_numbers=dn,
                           preferred_element_type=jnp.float32,
                           precision=lax.Precision.HIGHEST)


def _layer_body(h_ref, agg_ref, wa_ref, ba_ref, wb_ref, bb_ref, out_ref):
    z = h_ref[...] + agg_ref[0] + agg_ref[1]
    t = jnp.maximum(_bdot(z, wa_ref[...]) + ba_ref[...], 0.0)
    out_ref[...] = jnp.maximum(_bdot(t, wb_ref[...]) + bb_ref[...], 0.0)


def _tc_layer(h, aggp, wa, ba, wb, bb):
    n = h.shape[0]
    return pl.pallas_call(
        _layer_body,
        out_shape=jax.ShapeDtypeStruct((n, wb.shape[1]), jnp.float32),
    )(h, aggp, wa, ba.reshape(1, -1), wb, bb.reshape(1, -1))


def _final_body(h1_ref, agg_ref, w3_ref, b3_ref, w4_ref, b4_ref, wg_ref,
                bg_ref, batch_ref, wc1_ref, bc1_ref, wc2_ref, bc2_ref,
                wr1_ref, br1_ref, wr2_ref, br2_ref, cls_ref, reg_ref):
    n = h1_ref.shape[0]
    z = h1_ref[...] + agg_ref[0] + agg_ref[1]
    t = jnp.maximum(_bdot(z, w3_ref[...]) + b3_ref[...], 0.0)
    h2 = jnp.maximum(_bdot(t, w4_ref[...]) + b4_ref[...], 0.0)
    gate = _bdot(h2, wg_ref[...]) + bg_ref[...]

    batch = batch_ref[...]
    gids = lax.broadcasted_iota(jnp.int32, (n, NUM_GRAPHS), 1)
    mask = (batch == gids)
    maskf = mask.astype(jnp.float32)

    neg = jnp.float32(-1e30)
    gm = jnp.where(mask, gate, neg)
    m = jnp.max(gm, axis=0, keepdims=True)
    mb = jnp.sum(jnp.where(mask, m, 0.0), axis=1, keepdims=True)
    e = jnp.exp(gate - mb)
    dn = (((0,), (0,)), ((), ()))
    denom = _xdot(maskf, e, dn)
    denb = _xdot(maskf, denom)
    w = e / denb
    g = _xdot(maskf, w * h2, dn)

    c1 = jnp.maximum(_bdot(g, wc1_ref[...]) + bc1_ref[...], 0.0)
    cls_ref[...] = _bdot(c1, wc2_ref[...]) + bc2_ref[...]
    r1 = jnp.maximum(_bdot(g, wr1_ref[...]) + br1_ref[...], 0.0)
    reg_ref[...] = _bdot(r1, wr2_ref[...]) + br2_ref[...]


def _tc_final(h1, aggp, w3, b3, w4, b4, wg, bg, batch, wc1, bc1, wc2, bc2,
              wr1, br1, wr2, br2):
    c = wc2.shape[1]
    return pl.pallas_call(
        _final_body,
        out_shape=[jax.ShapeDtypeStruct((NUM_GRAPHS, c), jnp.float32),
                   jax.ShapeDtypeStruct((NUM_GRAPHS, 1), jnp.float32)],
    )(h1, aggp, w3, b3.reshape(1, -1), w4, b4.reshape(1, -1), wg,
      bg.reshape(1, -1), batch.reshape(-1, 1), wc1, bc1.reshape(1, -1),
      wc2, bc2.reshape(1, -1), wr1, br1.reshape(1, -1), wr2,
      br2.reshape(1, -1))


def kernel(x, edge_index, batch, W1, b1, W2, b2, W3, b3, W4, b4, Wg, bg,
           Wc1, bc1, Wc2, bc2, Wr1, br1, Wr2, br2):
    n, d = x.shape
    h = W1.shape[1]
    src = edge_index[0]
    dst = edge_index[1]
    src3, dst3, nch = _pad_edges(src, dst, n)

    agg1 = _make_edge_agg(n, d, nch)(x, src3, dst3)
    h1 = _tc_layer(x, agg1, W1, b1, W2, b2)
    agg2 = _make_edge_agg(n, h, nch)(h1, src3, dst3)
    cls, reg = _tc_final(h1, agg2, W3, b3, W4, b4, Wg, bg, batch,
                         Wc1, bc1, Wc2, bc2, Wr1, br1, Wr2, br2)
    return (cls, reg)


# speedup vs baseline: 2.1976x; 1.0978x over previous
import functools

import jax
import jax.numpy as jnp
from jax import lax
from jax.experimental import pallas as pl
from jax.experimental.pallas import tpu as pltpu
from jax.experimental.pallas import tpu_sc as plsc

NUM_GRAPHS = 64

_NC = 2
_NS = 16
_NW = _NC * _NS


_CHUNK = 50
_NBUF = 4
_NPAD = 8


def _pad_edges(src, dst, n_nodes):
    e = src.shape[0]
    epw = e // _NW
    grp = _CHUNK * _NBUF
    epw_pad = -(-epw // grp) * grp
    pad = epw_pad - epw
    src2 = src.reshape(_NW, epw)
    dst2 = dst.reshape(_NW, epw)
    if pad:
        src2 = jnp.pad(src2, ((0, 0), (0, pad)))
        dummy = (jnp.arange(pad, dtype=jnp.int32) % _NPAD) + n_nodes
        dst2 = jnp.concatenate(
            [dst2, jnp.broadcast_to(dummy, (_NW, pad))], axis=1)
    nch = epw_pad // _CHUNK
    return (src2.reshape(_NW, nch, _CHUNK), dst2.reshape(_NW, nch, _CHUNK),
            nch, _NPAD if pad else 0)


def _make_edge_agg(n_nodes, feat, nchunks, n_extra):
    n_acc = n_nodes + n_extra
    rpt = (n_nodes // _NS) // 8 * 8
    rem = n_nodes - rpt * _NS
    zrem = n_acc - rpt * _NS
    zrows = next(z for z in (16, 24, 48, 52, 104, 208)
                 if rpt % z == 0 and zrem <= z)
    assert rem % 8 == 0
    assert nchunks % _NBUF == 0
    ngroups = nchunks // _NBUF

    mesh = plsc.VectorSubcoreMesh(core_axis_name="c", subcore_axis_name="s")

    @functools.partial(
        pl.kernel,
        mesh=mesh,
        compiler_params=pltpu.CompilerParams(use_tc_tiling_on_sc=False),
        out_type=jax.ShapeDtypeStruct((_NC, n_nodes, feat), jnp.float32),
        scratch_types=(
            [pltpu.VMEM((nchunks, _CHUNK), jnp.int32),
             pltpu.VMEM((nchunks, _CHUNK), jnp.int32)]
            + [pltpu.VMEM((_CHUNK, feat), jnp.float32)] * _NBUF
            + [pltpu.VMEM((zrows, feat), jnp.float32),
               pltpu.VMEM_SHARED((n_acc, feat), jnp.float32)]
            + [pltpu.SemaphoreType.DMA] * (2 * _NBUF)
        ),
    )
    def edge_agg(y_hbm, src_hbm, dst_hbm, out_hbm, *scr):
        srcw, dstw = scr[0], scr[1]
        rows = scr[2:2 + _NBUF]
        zero_v, acc = scr[2 + _NBUF], scr[3 + _NBUF]
        gsem = scr[4 + _NBUF:4 + 2 * _NBUF]
        ssem = scr[4 + 2 * _NBUF:4 + 3 * _NBUF]

        cid = lax.axis_index("c")
        sid = lax.axis_index("s")
        wid = sid * _NC + cid
        is_last = sid == (_NS - 1)

        pltpu.sync_copy(src_hbm.at[wid], srcw)
        pltpu.sync_copy(dst_hbm.at[wid], dstw)

        @pl.loop(0, zrows)
        def _(i):
            @pl.loop(0, feat // 16)
            def _(j):
                zero_v[i, pl.ds(j * 16, 16)] = jnp.zeros((16,), jnp.float32)

        row0 = sid * rpt

        @pl.loop(0, rpt // zrows)
        def _(k):
            pltpu.sync_copy(zero_v, acc.at[pl.ds(row0 + k * zrows, zrows)])

        @pl.when(is_last)
        def _():
            pltpu.sync_copy(zero_v.at[pl.ds(0, zrem)],
                            acc.at[pl.ds(_NS * rpt, zrem)])

        plsc.subcore_barrier()

        def wait_on(sem, b):
            pltpu.make_async_copy(y_hbm.at[pl.ds(0, _CHUNK)], rows[b],
                                  sem).wait()

        for b in range(_NBUF):
            pltpu.async_copy(y_hbm.at[srcw.at[b]], rows[b], gsem[b])

        @pl.loop(0, ngroups - 1)
        def _(g):
            i0 = g * _NBUF
            for b in range(_NBUF):
                wait_on(gsem[b], b)
                pltpu.async_copy(rows[b], acc.at[dstw.at[i0 + b]],
                                 ssem[b], add=True)
            for b in range(_NBUF):
                wait_on(ssem[b], b)
                pltpu.async_copy(y_hbm.at[srcw.at[i0 + _NBUF + b]],
                                 rows[b], gsem[b])

        i0 = nchunks - _NBUF
        for b in range(_NBUF):
            wait_on(gsem[b], b)
            pltpu.async_copy(rows[b], acc.at[dstw.at[i0 + b]],
                             ssem[b], add=True)
        for b in range(_NBUF):
            wait_on(ssem[b], b)

        plsc.subcore_barrier()
        pltpu.sync_copy(acc.at[pl.ds(row0, rpt)],
                        out_hbm.at[cid, pl.ds(row0, rpt)])
        if rem:
            @pl.when(is_last)
            def _():
                pltpu.sync_copy(acc.at[pl.ds(_NS * rpt, rem)],
                                out_hbm.at[cid, pl.ds(_NS * rpt, rem)])

    return edge_agg


def _bdot(a, b):
    return jnp.dot(a.astype(jnp.bfloat16), b.astype(jnp.bfloat16),
                   preferred_element_type=jnp.float32)


def _xdot(a, b, dn=None):
    if dn is None:
        return jnp.dot(a, b, preferred_element_type=jnp.float32,
                       precision=lax.Precision.HIGHEST)
    return lax.dot_general(a, b, dimension_numbers=dn,
                           preferred_element_type=jnp.float32,
                           precision=lax.Precision.HIGHEST)


def _layer_body(h_ref, agg_ref, wa_ref, ba_ref, wb_ref, bb_ref, out_ref):
    z = h_ref[...] + agg_ref[0] + agg_ref[1]
    t = jnp.maximum(_bdot(z, wa_ref[...]) + ba_ref[...], 0.0)
    out_ref[...] = jnp.maximum(_bdot(t, wb_ref[...]) + bb_ref[...], 0.0)


def _tc_layer(h, aggp, wa, ba, wb, bb):
    n = h.shape[0]
    return pl.pallas_call(
        _layer_body,
        out_shape=jax.ShapeDtypeStruct((n, wb.shape[1]), jnp.float32),
    )(h, aggp, wa, ba.reshape(1, -1), wb, bb.reshape(1, -1))


def _final_body(h1_ref, agg_ref, w3_ref, b3_ref, w4_ref, b4_ref, wg_ref,
                bg_ref, batch_ref, wc1_ref, bc1_ref, wc2_ref, bc2_ref,
                wr1_ref, br1_ref, wr2_ref, br2_ref, cls_ref, reg_ref):
    n = h1_ref.shape[0]
    z = h1_ref[...] + agg_ref[0] + agg_ref[1]
    t = jnp.maximum(_bdot(z, w3_ref[...]) + b3_ref[...], 0.0)
    h2 = jnp.maximum(_bdot(t, w4_ref[...]) + b4_ref[...], 0.0)
    gate = _bdot(h2, wg_ref[...]) + bg_ref[...]

    batch = batch_ref[...]
    gids = lax.broadcasted_iota(jnp.int32, (n, NUM_GRAPHS), 1)
    mask = (batch == gids)
    maskf = mask.astype(jnp.float32)

    neg = jnp.float32(-1e30)
    gm = jnp.where(mask, gate, neg)
    m = jnp.max(gm, axis=0, keepdims=True)
    mb = jnp.sum(jnp.where(mask, m, 0.0), axis=1, keepdims=True)
    e = jnp.exp(gate - mb)
    dn = (((0,), (0,)), ((), ()))
    denom = _xdot(maskf, e, dn)
    denb = _xdot(maskf, denom)
    w = e / denb
    g = _xdot(maskf, w * h2, dn)

    c1 = jnp.maximum(_bdot(g, wc1_ref[...]) + bc1_ref[...], 0.0)
    cls_ref[...] = _bdot(c1, wc2_ref[...]) + bc2_ref[...]
    r1 = jnp.maximum(_bdot(g, wr1_ref[...]) + br1_ref[...], 0.0)
    reg_ref[...] = _bdot(r1, wr2_ref[...]) + br2_ref[...]


def _tc_final(h1, aggp, w3, b3, w4, b4, wg, bg, batch, wc1, bc1, wc2, bc2,
              wr1, br1, wr2, br2):
    c = wc2.shape[1]
    return pl.pallas_call(
        _final_body,
        out_shape=[jax.ShapeDtypeStruct((NUM_GRAPHS, c), jnp.float32),
                   jax.ShapeDtypeStruct((NUM_GRAPHS, 1), jnp.float32)],
    )(h1, aggp, w3, b3.reshape(1, -1), w4, b4.reshape(1, -1), wg,
      bg.reshape(1, -1), batch.reshape(-1, 1), wc1, bc1.reshape(1, -1),
      wc2, bc2.reshape(1, -1), wr1, br1.reshape(1, -1), wr2,
      br2.reshape(1, -1))


def kernel(x, edge_index, batch, W1, b1, W2, b2, W3, b3, W4, b4, Wg, bg,
           Wc1, bc1, Wc2, bc2, Wr1, br1, Wr2, br2):
    n, d = x.shape
    h = W1.shape[1]
    src = edge_index[0]
    dst = edge_index[1]
    src3, dst3, nch, nx = _pad_edges(src, dst, n)

    agg1 = _make_edge_agg(n, d, nch, nx)(x, src3, dst3)
    h1 = _tc_layer(x, agg1, W1, b1, W2, b2)
    agg2 = _make_edge_agg(n, h, nch, nx)(h1, src3, dst3)
    cls, reg = _tc_final(h1, agg2, W3, b3, W4, b4, Wg, bg, batch,
                         Wc1, bc1, Wc2, bc2, Wr1, br1, Wr2, br2)
    return (cls, reg)


# speedup vs baseline: 2.4536x; 1.1165x over previous
import functools

import jax
import jax.numpy as jnp
from jax import lax
from jax.experimental import pallas as pl
from jax.experimental.pallas import tpu as pltpu
from jax.experimental.pallas import tpu_sc as plsc

NUM_GRAPHS = 64

_NC = 2
_NS = 16
_NW = _NC * _NS


_CHUNK = 40
_NBUF = 5
_NPAD = 8


def _pad_edges(src, dst, n_nodes):
    e = src.shape[0]
    epw = e // _NW
    grp = _CHUNK * _NBUF
    epw_pad = -(-epw // grp) * grp
    pad = epw_pad - epw
    src2 = src.reshape(_NW, epw)
    dst2 = dst.reshape(_NW, epw)
    if pad:
        src2 = jnp.pad(src2, ((0, 0), (0, pad)))
        dummy = (jnp.arange(pad, dtype=jnp.int32) % _NPAD) + n_nodes
        dst2 = jnp.concatenate(
            [dst2, jnp.broadcast_to(dummy, (_NW, pad))], axis=1)
    nch = epw_pad // _CHUNK
    return (src2.reshape(_NW, nch, _CHUNK), dst2.reshape(_NW, nch, _CHUNK),
            nch, _NPAD if pad else 0)


def _make_edge_agg(n_nodes, feat, nchunks, n_extra):
    n_acc = n_nodes + n_extra
    rpt = (n_nodes // _NS) // 8 * 8
    rem = n_nodes - rpt * _NS
    zrem = n_acc - rpt * _NS
    zrows = next(z for z in (16, 24, 48, 52, 104, 208)
                 if rpt % z == 0 and zrem <= z)
    assert rem % 8 == 0
    assert nchunks % _NBUF == 0
    ngroups = nchunks // _NBUF

    mesh = plsc.VectorSubcoreMesh(core_axis_name="c", subcore_axis_name="s")

    @functools.partial(
        pl.kernel,
        mesh=mesh,
        compiler_params=pltpu.CompilerParams(use_tc_tiling_on_sc=False),
        out_type=jax.ShapeDtypeStruct((_NC, n_nodes, feat), jnp.float32),
        scratch_types=(
            [pltpu.VMEM((nchunks, _CHUNK), jnp.int32),
             pltpu.VMEM((nchunks, _CHUNK), jnp.int32)]
            + [pltpu.VMEM((_CHUNK, feat), jnp.float32)] * _NBUF
            + [pltpu.VMEM((zrows, feat), jnp.float32),
               pltpu.VMEM_SHARED((n_acc, feat), jnp.float32)]
            + [pltpu.SemaphoreType.DMA] * (2 * _NBUF)
        ),
    )
    def edge_agg(y_hbm, src_hbm, dst_hbm, out_hbm, *scr):
        srcw, dstw = scr[0], scr[1]
        rows = scr[2:2 + _NBUF]
        zero_v, acc = scr[2 + _NBUF], scr[3 + _NBUF]
        gsem = scr[4 + _NBUF:4 + 2 * _NBUF]
        ssem = scr[4 + 2 * _NBUF:4 + 3 * _NBUF]

        cid = lax.axis_index("c")
        sid = lax.axis_index("s")
        wid = sid * _NC + cid
        is_last = sid == (_NS - 1)

        pltpu.sync_copy(src_hbm.at[wid], srcw)
        pltpu.sync_copy(dst_hbm.at[wid], dstw)

        @pl.loop(0, zrows)
        def _(i):
            @pl.loop(0, feat // 16)
            def _(j):
                zero_v[i, pl.ds(j * 16, 16)] = jnp.zeros((16,), jnp.float32)

        row0 = sid * rpt

        @pl.loop(0, rpt // zrows)
        def _(k):
            pltpu.sync_copy(zero_v, acc.at[pl.ds(row0 + k * zrows, zrows)])

        @pl.when(is_last)
        def _():
            pltpu.sync_copy(zero_v.at[pl.ds(0, zrem)],
                            acc.at[pl.ds(_NS * rpt, zrem)])

        plsc.subcore_barrier()

        def wait_on(sem, b):
            pltpu.make_async_copy(y_hbm.at[pl.ds(0, _CHUNK)], rows[b],
                                  sem).wait()

        for b in range(_NBUF):
            pltpu.async_copy(y_hbm.at[srcw.at[b]], rows[b], gsem[b])

        @pl.loop(0, ngroups - 1)
        def _(g):
            i0 = g * _NBUF
            for b in range(_NBUF):
                wait_on(gsem[b], b)
                pltpu.async_copy(rows[b], acc.at[dstw.at[i0 + b]],
                                 ssem[b], add=True)
            for b in range(_NBUF):
                wait_on(ssem[b], b)
                pltpu.async_copy(y_hbm.at[srcw.at[i0 + _NBUF + b]],
                                 rows[b], gsem[b])

        i0 = nchunks - _NBUF
        for b in range(_NBUF):
            wait_on(gsem[b], b)
            pltpu.async_copy(rows[b], acc.at[dstw.at[i0 + b]],
                             ssem[b], add=True)
        for b in range(_NBUF):
            wait_on(ssem[b], b)

        plsc.subcore_barrier()
        pltpu.sync_copy(acc.at[pl.ds(row0, rpt)],
                        out_hbm.at[cid, pl.ds(row0, rpt)])
        if rem:
            @pl.when(is_last)
            def _():
                pltpu.sync_copy(acc.at[pl.ds(_NS * rpt, rem)],
                                out_hbm.at[cid, pl.ds(_NS * rpt, rem)])

    return edge_agg


def _bdot(a, b):
    return jnp.dot(a.astype(jnp.bfloat16), b.astype(jnp.bfloat16),
                   preferred_element_type=jnp.float32)


def _xdot(a, b, dn=None):
    if dn is None:
        return jnp.dot(a, b, preferred_element_type=jnp.float32,
                       precision=lax.Precision.HIGHEST)
    return lax.dot_general(a, b, dimension_numbers=dn,
                           preferred_element_type=jnp.float32,
                           precision=lax.Precision.HIGHEST)


def _layer_body(h_ref, agg_ref, wa_ref, ba_ref, wb_ref, bb_ref, out_ref):
    z = h_ref[...] + agg_ref[0] + agg_ref[1]
    t = jnp.maximum(_bdot(z, wa_ref[...]) + ba_ref[...], 0.0)
    out_ref[...] = jnp.maximum(_bdot(t, wb_ref[...]) + bb_ref[...], 0.0)


def _tc_layer(h, aggp, wa, ba, wb, bb):
    n = h.shape[0]
    return pl.pallas_call(
        _layer_body,
        out_shape=jax.ShapeDtypeStruct((n, wb.shape[1]), jnp.float32),
    )(h, aggp, wa, ba.reshape(1, -1), wb, bb.reshape(1, -1))


def _final_body(h1_ref, agg_ref, w3_ref, b3_ref, w4_ref, b4_ref, wg_ref,
                bg_ref, batch_ref, wc1_ref, bc1_ref, wc2_ref, bc2_ref,
                wr1_ref, br1_ref, wr2_ref, br2_ref, cls_ref, reg_ref):
    n = h1_ref.shape[0]
    z = h1_ref[...] + agg_ref[0] + agg_ref[1]
    t = jnp.maximum(_bdot(z, w3_ref[...]) + b3_ref[...], 0.0)
    h2 = jnp.maximum(_bdot(t, w4_ref[...]) + b4_ref[...], 0.0)
    gate = _bdot(h2, wg_ref[...]) + bg_ref[...]

    batch = batch_ref[...]
    gids = lax.broadcasted_iota(jnp.int32, (n, NUM_GRAPHS), 1)
    mask = (batch == gids)
    maskf = mask.astype(jnp.float32)

    neg = jnp.float32(-1e30)
    gm = jnp.where(mask, gate, neg)
    m = jnp.max(gm, axis=0, keepdims=True)
    mb = jnp.sum(jnp.where(mask, m, 0.0), axis=1, keepdims=True)
    e = jnp.exp(gate - mb)
    dn = (((0,), (0,)), ((), ()))
    denom = _xdot(maskf, e, dn)
    denb = _xdot(maskf, denom)
    w = e / denb
    g = _xdot(maskf, w * h2, dn)

    c1 = jnp.maximum(_bdot(g, wc1_ref[...]) + bc1_ref[...], 0.0)
    cls_ref[...] = _bdot(c1, wc2_ref[...]) + bc2_ref[...]
    r1 = jnp.maximum(_bdot(g, wr1_ref[...]) + br1_ref[...], 0.0)
    reg_ref[...] = _bdot(r1, wr2_ref[...]) + br2_ref[...]


def _tc_final(h1, aggp, w3, b3, w4, b4, wg, bg, batch, wc1, bc1, wc2, bc2,
              wr1, br1, wr2, br2):
    c = wc2.shape[1]
    return pl.pallas_call(
        _final_body,
        out_shape=[jax.ShapeDtypeStruct((NUM_GRAPHS, c), jnp.float32),
                   jax.ShapeDtypeStruct((NUM_GRAPHS, 1), jnp.float32)],
    )(h1, aggp, w3, b3.reshape(1, -1), w4, b4.reshape(1, -1), wg,
      bg.reshape(1, -1), batch.reshape(-1, 1), wc1, bc1.reshape(1, -1),
      wc2, bc2.reshape(1, -1), wr1, br1.reshape(1, -1), wr2,
      br2.reshape(1, -1))


def kernel(x, edge_index, batch, W1, b1, W2, b2, W3, b3, W4, b4, Wg, bg,
           Wc1, bc1, Wc2, bc2, Wr1, br1, Wr2, br2):
    n, d = x.shape
    h = W1.shape[1]
    src = edge_index[0]
    dst = edge_index[1]
    src3, dst3, nch, nx = _pad_edges(src, dst, n)

    agg1 = _make_edge_agg(n, d, nch, nx)(x, src3, dst3)
    h1 = _tc_layer(x, agg1, W1, b1, W2, b2)
    agg2 = _make_edge_agg(n, h, nch, nx)(h1, src3, dst3)
    cls, reg = _tc_final(h1, agg2, W3, b3, W4, b4, Wg, bg, batch,
                         Wc1, bc1, Wc2, bc2, Wr1, br1, Wr2, br2)
    return (cls, reg)
